# baseline (speedup 1.0000x reference)
import functools

import jax
import jax.numpy as jnp
from jax.experimental import pallas as pl
from jax.experimental.pallas import tpu as pltpu

_NCHUNK = 8
_T = 128


def _dma_body(
    scal_ref, cache_hbm, act1_hbm, act2_hbm, patch_hbm, pend_hbm, out_hbm, sem, *, M, B
):
    idx = scal_ref[0]
    delta = scal_ref[1]
    l1 = scal_ref[2]
    o2 = scal_ref[3]
    mend = (M // _T) * _T

    nrow = cache_hbm.shape[0] // 16
    copies = []
    for k in range(nrow):
        copies.append(
            pltpu.make_async_copy(
                cache_hbm.at[pl.ds(16 * k, 16), :],
                out_hbm.at[pl.ds(16 * k, 16), :],
                sem,
            )
        )
    for cp in copies:
        cp.start()
    for cp in copies:
        cp.wait()

    def seg_plan(s, e, off):
        s1 = ((s + _T - 1) // _T) * _T
        e1 = jnp.minimum((e // _T) * _T, mend)
        n = jnp.maximum(e1 - s1, 0)
        return s1, off, n

    plans = [
        (seg_plan(idx, idx + l1, _T + delta - idx), act1_hbm),
        (seg_plan(0, B - l1, _T + o2 + l1), act2_hbm),
    ]

    def seg_emit(do_start):
        for (s1, off, n), ref in plans:
            cur, rem = s1, n
            for bit in reversed(range(8)):
                sz = _T << bit
                take = rem >= sz

                @pl.when(take)
                def _(cur=cur, off=off, sz=sz, ref=ref):
                    cp = pltpu.make_async_copy(
                        ref.at[:, pl.ds(pl.multiple_of(cur + off, _T), sz)],
                        out_hbm.at[:, pl.ds(pl.multiple_of(cur, _T), sz)],
                        sem,
                    )
                    cp.start() if do_start else cp.wait()

                step = jnp.where(take, sz, 0)
                cur, rem = cur + step, rem - step

    seg_emit(True)
    patch_cps = [
        pltpu.make_async_copy(
            patch_hbm.at[:, pl.ds(_T * k, _T)],
            out_hbm.at[:, pl.ds(pl.multiple_of(scal_ref[4 + k], _T), _T)],
            sem,
        )
        for k in range(4)
    ]
    pend_cp = pltpu.make_async_copy(pend_hbm, out_hbm.at[:, pl.ds(mend, M - mend)], sem)
    for cp in patch_cps:
        cp.start()
    pend_cp.start()
    seg_emit(False)
    for cp in patch_cps:
        cp.wait()
    pend_cp.wait()


def kernel(activations, cache, n_valid, index):
    M, N = cache.shape
    B = activations.shape[0]
    assert B % _T == 0
    mend = (M // _T) * _T

    idx = jnp.asarray(index, jnp.int32) % M
    cache_t = jax.lax.bitcast_convert_type(jnp.transpose(cache), jnp.int16)
    act_t = jax.lax.bitcast_convert_type(
        jnp.transpose(activations).astype(cache.dtype), jnp.int16
    )

    delta = idx % _T
    o2 = (delta - M % _T) % _T
    L = B + 3 * _T
    act1 = jax.lax.dynamic_update_slice(
        jnp.zeros((N, L), jnp.int16), act_t, (0, _T + delta)
    )
    act2 = jax.lax.dynamic_update_slice(
        jnp.zeros((N, L), jnp.int16), act_t, (0, _T + o2)
    )

    l1 = jnp.minimum(B, M - idx)
    e2 = (idx + B) % M
    d0s = jnp.stack(
        [
            jnp.minimum((idx // _T) * _T, mend - _T),
            jnp.minimum(((idx + l1) // _T) * _T, mend - _T),
            jnp.zeros((), jnp.int32),
            jnp.minimum((e2 // _T) * _T, mend - _T),
        ]
    )

    def merged_cols(d0, width):
        pv = jax.lax.dynamic_slice(cache_t, (0, d0), (N, width))
        use_wrap = d0 + width - 1 < idx
        b1 = jnp.clip(_T + delta + d0 - idx, 0, L - width)
        b2 = jnp.clip(_T + o2 + l1 + d0, 0, L - width)
        av1 = jax.lax.dynamic_slice(act1, (0, b1), (N, width))
        av2 = jax.lax.dynamic_slice(act2, (0, b2), (N, width))
        av = jnp.where(use_wrap, av2, av1)
        offw = (d0 + jnp.arange(width, dtype=jnp.int32) - idx) % M
        return jnp.where((offw < B)[None, :], av, pv)

    patches = jnp.concatenate([merged_cols(d0s[k], _T) for k in range(4)], axis=1)
    pend = merged_cols(jnp.asarray(mend, jnp.int32), M - mend)

    scal = jnp.stack([idx, delta, l1, o2, d0s[0], d0s[1], d0s[2], d0s[3]])

    grid_spec = pltpu.PrefetchScalarGridSpec(
        num_scalar_prefetch=1,
        grid=(1,),
        in_specs=[pl.BlockSpec(memory_space=pltpu.MemorySpace.HBM)] * 5,
        out_specs=pl.BlockSpec(memory_space=pltpu.MemorySpace.HBM),
        scratch_shapes=[pltpu.SemaphoreType.DMA],
    )
    out_t = pl.pallas_call(
        functools.partial(_dma_body, M=M, B=B),
        grid_spec=grid_spec,
        out_shape=jax.ShapeDtypeStruct((N, M), jnp.int16),
    )(scal, cache_t, act1, act2, patches, pend)

    new_cache = jnp.transpose(jax.lax.bitcast_convert_type(out_t, cache.dtype))
    new_n_valid = jnp.minimum(jnp.asarray(n_valid) + B, M)
    new_index = (jnp.asarray(index) + B) % M
    return (new_cache, new_n_valid, new_index)


# speedup vs baseline: 5.3501x; 5.3501x over previous
import functools

import jax
import jax.numpy as jnp
from jax.experimental import pallas as pl
from jax.experimental.pallas import tpu as pltpu

_PAD = 16


def _dma_body(scal_ref, cache_hbm, act_hbm, patch_hbm, out_hbm, sem, *, M, B):
    idx = scal_ref[0]
    delta = scal_ref[1]
    l1 = scal_ref[2]

    def seg_plan(s, e, src_at_s):
        s8 = ((s + 7) // 8) * 8
        e8 = (e // 8) * 8
        n = jnp.maximum(e8 - s8, 0)
        return s8, src_at_s + (s8 - s), n

    plans = [
        seg_plan(idx, idx + l1, _PAD + delta),
        seg_plan(0, B - l1, _PAD + delta + l1),
    ]

    def seg_emit(do_start):
        for s8, src0, n in plans:
            cur_d, cur_s, rem = s8, src0, n
            for bit in reversed(range(12)):
                sz = 8 << bit
                take = rem >= sz

                @pl.when(take)
                def _(cur_d=cur_d, cur_s=cur_s, sz=sz):
                    cp = pltpu.make_async_copy(
                        act_hbm.at[pl.ds(pl.multiple_of(cur_s, 8), sz)],
                        out_hbm.at[pl.ds(pl.multiple_of(cur_d, 8), sz)],
                        sem,
                    )
                    cp.start() if do_start else cp.wait()

                step = jnp.where(take, sz, 0)
                cur_d, cur_s, rem = cur_d + step, cur_s + step, rem - step

    seg_emit(True)
    patch_cps = [
        pltpu.make_async_copy(
            patch_hbm.at[pl.ds(8 * k, 8)],
            out_hbm.at[pl.ds(pl.multiple_of(scal_ref[4 + k], 8), 8)],
            sem,
        )
        for k in range(4)
    ]
    for cp in patch_cps:
        cp.start()
    seg_emit(False)
    for cp in patch_cps:
        cp.wait()


def kernel(activations, cache, n_valid, index):
    M, N = cache.shape
    B = activations.shape[0]

    idx = jnp.asarray(index, jnp.int32) % M
    act16 = jax.lax.bitcast_convert_type(activations.astype(cache.dtype), jnp.int16)
    cache_u = jax.lax.bitcast_convert_type(cache, jnp.int16)
    delta = idx % 8
    L = B + 3 * _PAD
    act_pad = jax.lax.dynamic_update_slice(
        jnp.zeros((L, N), jnp.int16), act16, (_PAD + delta, 0)
    )

    l1 = jnp.minimum(B, M - idx)
    e2 = (idx + B) % M
    d0s = jnp.stack(
        [
            (idx // 8) * 8,
            jnp.minimum(((idx + l1) // 8) * 8, M - 8),
            jnp.zeros((), jnp.int32),
            jnp.minimum((e2 // 8) * 8, M - 8),
        ]
    )

    def patch(d0):
        pv = jax.lax.dynamic_slice(cache_u, (d0, 0), (8, N))
        base = _PAD + delta + d0 - idx + jnp.where(d0 + 7 < idx, M, 0)
        av = jax.lax.dynamic_slice(
            act_pad, (jnp.clip(base, 0, L - 8), 0), (8, N)
        )
        offw = (d0 + jnp.arange(8, dtype=jnp.int32) - idx) % M
        return jnp.where((offw < B)[:, None], av, pv)

    patches = jnp.concatenate([patch(d0s[k]) for k in range(4)], axis=0)

    scal = jnp.stack([idx, delta, l1, e2, d0s[0], d0s[1], d0s[2], d0s[3]])

    grid_spec = pltpu.PrefetchScalarGridSpec(
        num_scalar_prefetch=1,
        grid=(1,),
        in_specs=[
            pl.BlockSpec(memory_space=pltpu.MemorySpace.HBM),
            pl.BlockSpec(memory_space=pltpu.MemorySpace.HBM),
            pl.BlockSpec(memory_space=pltpu.MemorySpace.HBM),
        ],
        out_specs=pl.BlockSpec(memory_space=pltpu.MemorySpace.HBM),
        scratch_shapes=[pltpu.SemaphoreType.DMA],
    )
    out_u = pl.pallas_call(
        functools.partial(_dma_body, M=M, B=B),
        grid_spec=grid_spec,
        out_shape=jax.ShapeDtypeStruct((M, N), jnp.int16),
        input_output_aliases={1: 0},
    )(scal, cache_u, act_pad, patches)

    new_cache = jax.lax.bitcast_convert_type(out_u, cache.dtype)
    new_n_valid = jnp.minimum(jnp.asarray(n_valid) + B, M)
    new_index = (jnp.asarray(index) + B) % M
    return (new_cache, new_n_valid, new_index)
